# baseline (speedup 1.0000x reference)
import jax
import jax.numpy as jnp
from jax import lax
from jax.experimental import pallas as pl
from jax.experimental.pallas import tpu as pltpu
from jax.experimental.pallas import tpu_sc as plsc

_MERGE_SIZE = 4
_KERNEL = 2
_MAX_TOKENS = 780 // _MERGE_SIZE

_B = 16
_L = 2048
_D = 1024
_LANES = 16
_CT = 8
_NT = _B * _MAX_TOKENS
_NW = 32
_DV = _D // _LANES


def _sc_body(hs_ref, grid_ref, out_ref, attn_ref,
             grid_v, w2_v, wp_v, m_v, idx_v, rows_v, out_v, zero_v,
             scale_v, attn_v, semg0, semg1, semo0, semo1):
    semg = (semg0, semg1)
    semo = (semo0, semo1)
    wid = lax.axis_index("s") * 2 + lax.axis_index("c")
    lanes = lax.iota(jnp.int32, _LANES)
    tok = lanes % _CT
    jbit = lanes // _CT

    pltpu.sync_copy(grid_ref, grid_v)
    h = plsc.load_gather(grid_v, [lanes * 3 + 1])
    w = plsc.load_gather(grid_v, [lanes * 3 + 2])
    w2 = w // 2
    wp = w2 // _KERNEL
    hp = (h // 2) // _KERNEL
    w2_v[...] = w2
    wp_v[...] = wp
    m_v[...] = hp * wp

    zf = jnp.zeros((_LANES,), jnp.float32)

    def zero_body(tt, carry):
        for vi in range(_DV):
            zero_v[tt, pl.ds(vi * _LANES, _LANES)] = zf
        return carry

    lax.fori_loop(0, _CT, zero_body, 0)

    n = (_NT // _CT - wid + _NW - 1) // _NW

    bv = (wid % 2) * _CT + tok
    ml = plsc.load_gather(m_v, [bv])
    w2l = plsc.load_gather(w2_v, [bv])
    wpl = plsc.load_gather(wp_v, [bv])
    off2 = bv * (_L // 2)
    maxm = jnp.max(ml)
    pbase = wid // 2

    def chunk_params(i):
        t0 = pl.multiple_of((wid + i * _NW) * _CT, _CT)
        ps = pbase + i * _LANES
        p = jnp.full((_LANES,), ps, jnp.int32)
        return t0, p, p < ml, ps < maxm

    def fire_gather(i, buf):
        _, p, _, hv = chunk_params(i)

        @pl.when(hv)
        def _():
            r = p // wpl
            c = p - r * wpl
            pr = r * w2l + c
            lim = _L // 2 - 1
            idx_v[buf, :] = jnp.minimum(pr + jbit * (w2l // 2), lim) + off2
            pltpu.async_copy(hs_ref.at[idx_v.at[buf]], rows_v.at[buf],
                             semg[buf])

    def process(i, buf):
        t0, p, valid, hv = chunk_params(i)

        @pl.when(i >= 2)
        def _():
            pltpu.make_async_copy(out_v.at[buf],
                                  out_ref.at[pl.ds(0, _CT)],
                                  semo[buf]).wait()

        @pl.when(hv)
        def _():
            pltpu.make_async_copy(hs_ref.at[idx_v.at[buf]], rows_v.at[buf],
                                  semg[buf]).wait()
            scale_v[...] = jnp.where(valid, jnp.float32(0.25),
                                     jnp.float32(0.0))

            def tok_body(tt, c2):
                s = plsc.load_gather(
                    scale_v, [jnp.full((_LANES,), tt, jnp.int32)])
                for vi in range(_DV):
                    sl = pl.ds(vi * _LANES, _LANES)
                    sr = pl.ds(_D + vi * _LANES, _LANES)
                    acc = ((rows_v[buf, tt, sl] + rows_v[buf, tt, sr])
                           + (rows_v[buf, tt + _CT, sl]
                              + rows_v[buf, tt + _CT, sr]))
                    out_v[buf, tt, sl] = acc * s
                return c2

            lax.fori_loop(0, _CT, tok_body, 0)
            pltpu.async_copy(out_v.at[buf], out_ref.at[pl.ds(t0, _CT)],
                             semo[buf])

        @pl.when(jnp.logical_not(hv))
        def _():
            pltpu.async_copy(zero_v, out_ref.at[pl.ds(t0, _CT)], semo[buf])

    fire_gather(0, 0)

    def outer(i2, carry):
        for buf in (0, 1):
            i = i2 * 2 + buf

            @pl.when(i < n)
            def _():
                @pl.when(i + 1 < n)
                def _():
                    fire_gather(i + 1, 1 - buf)

                process(i, buf)

        return carry

    lax.fori_loop(0, (n + 1) // 2, outer, 0)

    for buf in (0, 1):
        pltpu.make_async_copy(out_v.at[buf], out_ref.at[pl.ds(0, _CT)],
                              semo[buf]).wait()

    nf = _NT // _LANES
    nmine = (nf - wid + _NW - 1) // _NW

    def attn_body(i, carry):
        g = wid + i * _NW
        t0 = pl.multiple_of(g * _LANES, _LANES)
        t = t0 + lanes
        b = t // _MAX_TOKENS
        pp = t - b * _MAX_TOKENS
        ml = plsc.load_gather(m_v, [b])
        attn_v[...] = jnp.where(pp < ml, jnp.float32(1.0), jnp.float32(0.0))
        pltpu.sync_copy(attn_v, attn_ref.at[pl.ds(t0, _LANES)])
        return carry

    lax.fori_loop(0, nmine, attn_body, 0)


def _build():
    mesh = plsc.VectorSubcoreMesh(core_axis_name="c", subcore_axis_name="s")
    return pl.kernel(
        _sc_body,
        out_type=[
            jax.ShapeDtypeStruct((_NT, _D), jnp.float32),
            jax.ShapeDtypeStruct((_NT,), jnp.float32),
        ],
        mesh=mesh,
        compiler_params=pltpu.CompilerParams(needs_layout_passes=False),
        scratch_types=[
            pltpu.VMEM((_B * 3,), jnp.int32),
            pltpu.VMEM((_LANES,), jnp.int32),
            pltpu.VMEM((_LANES,), jnp.int32),
            pltpu.VMEM((_LANES,), jnp.int32),
            pltpu.VMEM((2, 2 * _CT), jnp.int32),
            pltpu.VMEM((2, 2 * _CT, 2 * _D), jnp.float32),
            pltpu.VMEM((2, _CT, _D), jnp.float32),
            pltpu.VMEM((_CT, _D), jnp.float32),
            pltpu.VMEM((_LANES,), jnp.float32),
            pltpu.VMEM((_LANES,), jnp.float32),
            pltpu.SemaphoreType.DMA,
            pltpu.SemaphoreType.DMA,
            pltpu.SemaphoreType.DMA,
            pltpu.SemaphoreType.DMA,
        ],
    )


def kernel(hidden_states, attention_mask, image_grid_thw):
    B, L, D = hidden_states.shape
    assert (B, L, D) == (_B, _L, _D)
    hs_flat = hidden_states.reshape(B * L // 2, 2 * D)
    grid_flat = jnp.asarray(image_grid_thw).astype(jnp.int32).reshape(-1)
    out_pm, attn_flat = _build()(hs_flat, grid_flat)
    outputs = out_pm.reshape(_MAX_TOKENS, B, D).transpose(1, 0, 2)
    outputs_attention = attn_flat.reshape(B, _MAX_TOKENS).astype(
        attention_mask.dtype)
    return outputs, outputs_attention


# speedup vs baseline: 3.2622x; 3.2622x over previous
import jax
import jax.numpy as jnp
from jax import lax
from jax.experimental import pallas as pl
from jax.experimental.pallas import tpu as pltpu
from jax.experimental.pallas import tpu_sc as plsc

_MERGE_SIZE = 4
_KERNEL = 2
_MAX_TOKENS = 780 // _MERGE_SIZE

_B = 16
_L = 2048
_D = 1024
_LANES = 16
_CT = 8
_NT = _B * _MAX_TOKENS
_NW = 32
_DV = _D // _LANES


def _sc_body(hs_ref, grid_ref, out_ref, attn_ref,
             grid_v, w2_v, wp_v, m_v, idx_v, rows_v, out_v,
             scale_v, attn_v, semg0, semg1, semg2, semo0, semo1, semo2):
    semg = (semg0, semg1, semg2)
    semo = (semo0, semo1, semo2)
    wid = lax.axis_index("s") * 2 + lax.axis_index("c")
    lanes = lax.iota(jnp.int32, _LANES)
    tok = lanes % _CT
    jbit = lanes // _CT

    pltpu.sync_copy(grid_ref, grid_v)
    h = plsc.load_gather(grid_v, [lanes * 3 + 1])
    w = plsc.load_gather(grid_v, [lanes * 3 + 2])
    w2 = w // 2
    wp = w2 // _KERNEL
    hp = (h // 2) // _KERNEL
    w2_v[...] = w2
    wp_v[...] = wp
    m_v[...] = hp * wp

    zf = jnp.zeros((_LANES,), jnp.float32)

    n = (_NT // _CT - wid + _NW - 1) // _NW

    bv = (wid % 2) * _CT + tok
    ml = plsc.load_gather(m_v, [bv])
    w2l = plsc.load_gather(w2_v, [bv])
    wpl = plsc.load_gather(wp_v, [bv])
    off = bv * _L
    maxm = jnp.max(ml)
    pbase = wid // 2

    def chunk_params(i):
        t0 = pl.multiple_of((wid + i * _NW) * _CT, _CT)
        ps = pbase + i * _LANES
        p = jnp.full((_LANES,), ps, jnp.int32)
        return t0, p, p < ml, ps < maxm

    def fire_gather(i, buf):
        _, p, _, hv = chunk_params(i)

        @pl.when(hv)
        def _():
            r = p // wpl
            c = p - r * wpl
            base = 2 * r * w2l + 2 * c
            lim = _L - 1
            idx_v[buf, pl.ds(0, _LANES)] = (
                jnp.minimum(base + jbit, lim) + off)
            idx_v[buf, pl.ds(_LANES, _LANES)] = (
                jnp.minimum(base + w2l + jbit, lim) + off)
            pltpu.async_copy(hs_ref.at[idx_v.at[buf]], rows_v.at[buf],
                             semg[buf])

    def process(i, buf):
        t0, p, valid, hv = chunk_params(i)

        @pl.when(i >= 3)
        def _():
            pltpu.make_async_copy(out_v.at[buf],
                                  out_ref.at[pl.ds(0, _CT)],
                                  semo[buf]).wait()

        @pl.when(hv)
        def _():
            pltpu.make_async_copy(hs_ref.at[idx_v.at[buf]], rows_v.at[buf],
                                  semg[buf]).wait()
            scale_v[...] = jnp.where(valid, jnp.float32(0.25),
                                     jnp.float32(0.0))

            def tok_body(tt, c2):
                s = plsc.load_gather(
                    scale_v, [jnp.full((_LANES,), tt, jnp.int32)])
                for vi in range(_DV):
                    sl = pl.ds(vi * _LANES, _LANES)
                    acc = ((rows_v[buf, tt, sl]
                            + rows_v[buf, tt + _CT, sl])
                           + (rows_v[buf, tt + 2 * _CT, sl]
                              + rows_v[buf, tt + 3 * _CT, sl]))
                    out_v[buf, tt, sl] = acc * s
                return c2

            lax.fori_loop(0, _CT, tok_body, 0)
            pltpu.async_copy(out_v.at[buf], out_ref.at[pl.ds(t0, _CT)],
                             semo[buf])

        @pl.when(jnp.logical_not(hv))
        def _():
            def zbody(tt, c2):
                for vi in range(_DV):
                    out_v[buf, tt, pl.ds(vi * _LANES, _LANES)] = zf
                return c2

            lax.fori_loop(0, _CT, zbody, 0)
            pltpu.async_copy(out_v.at[buf], out_ref.at[pl.ds(t0, _CT)],
                             semo[buf])

    fire_gather(0, 0)
    fire_gather(1, 1)

    def outer(i3, carry):
        for buf in (0, 1, 2):
            i = i3 * 3 + buf

            @pl.when(i < n)
            def _():
                @pl.when(i + 2 < n)
                def _():
                    fire_gather(i + 2, (buf + 2) % 3)

                process(i, buf)

        return carry

    lax.fori_loop(0, (n + 2) // 3, outer, 0)

    for buf in (0, 1, 2):
        pltpu.make_async_copy(out_v.at[buf], out_ref.at[pl.ds(0, _CT)],
                              semo[buf]).wait()

    nf = _NT // _LANES
    nmine = (nf - wid + _NW - 1) // _NW

    def attn_body(i, carry):
        g = wid + i * _NW
        t0 = pl.multiple_of(g * _LANES, _LANES)
        t = t0 + lanes
        b = t // _MAX_TOKENS
        pp = t - b * _MAX_TOKENS
        ml = plsc.load_gather(m_v, [b])
        attn_v[...] = jnp.where(pp < ml, jnp.float32(1.0), jnp.float32(0.0))
        pltpu.sync_copy(attn_v, attn_ref.at[pl.ds(t0, _LANES)])
        return carry

    lax.fori_loop(0, nmine, attn_body, 0)


def _build():
    mesh = plsc.VectorSubcoreMesh(core_axis_name="c", subcore_axis_name="s")
    return pl.kernel(
        _sc_body,
        out_type=[
            jax.ShapeDtypeStruct((_NT, _D), jnp.float32),
            jax.ShapeDtypeStruct((_NT,), jnp.float32),
        ],
        mesh=mesh,
        compiler_params=pltpu.CompilerParams(needs_layout_passes=False),
        scratch_types=[
            pltpu.VMEM((_B * 3,), jnp.int32),
            pltpu.VMEM((_LANES,), jnp.int32),
            pltpu.VMEM((_LANES,), jnp.int32),
            pltpu.VMEM((_LANES,), jnp.int32),
            pltpu.VMEM((3, 4 * _CT), jnp.int32),
            pltpu.VMEM((3, 4 * _CT, _D), jnp.float32),
            pltpu.VMEM((3, _CT, _D), jnp.float32),
            pltpu.VMEM((_LANES,), jnp.float32),
            pltpu.VMEM((_LANES,), jnp.float32),
            pltpu.SemaphoreType.DMA,
            pltpu.SemaphoreType.DMA,
            pltpu.SemaphoreType.DMA,
            pltpu.SemaphoreType.DMA,
            pltpu.SemaphoreType.DMA,
            pltpu.SemaphoreType.DMA,
        ],
    )


def kernel(hidden_states, attention_mask, image_grid_thw):
    B, L, D = hidden_states.shape
    assert (B, L, D) == (_B, _L, _D)
    hs_flat = hidden_states.reshape(B * L, D)
    grid_flat = jnp.asarray(image_grid_thw).astype(jnp.int32).reshape(-1)
    out_pm, attn_flat = _build()(hs_flat, grid_flat)
    outputs = out_pm.reshape(_MAX_TOKENS, B, D).transpose(1, 0, 2)
    outputs_attention = attn_flat.reshape(B, _MAX_TOKENS).astype(
        attention_mask.dtype)
    return outputs, outputs_attention


# speedup vs baseline: 3.5756x; 1.0961x over previous
import jax
import jax.numpy as jnp
from jax import lax
from jax.experimental import pallas as pl
from jax.experimental.pallas import tpu as pltpu
from jax.experimental.pallas import tpu_sc as plsc

_MERGE_SIZE = 4
_KERNEL = 2
_MAX_TOKENS = 780 // _MERGE_SIZE

_B = 16
_L = 2048
_D = 1024
_LANES = 16
_CT = 8
_NT = _B * _MAX_TOKENS
_NW = 32
_DV = _D // _LANES


def _sc_body(hs_ref, grid_ref, out_ref, attn_ref,
             grid_v, w2_v, wp_v, m_v, idx_v, rows_v, out_v, zero_v,
             scale_v, attn_v, semg0, semg1, semo0, semo1):
    semg = (semg0, semg1)
    semo = (semo0, semo1)
    wid = lax.axis_index("s") * 2 + lax.axis_index("c")
    lanes = lax.iota(jnp.int32, _LANES)
    tok = lanes % _CT
    jbit = lanes // _CT

    pltpu.sync_copy(grid_ref, grid_v)
    h = plsc.load_gather(grid_v, [lanes * 3 + 1])
    w = plsc.load_gather(grid_v, [lanes * 3 + 2])
    w2 = w // 2
    wp = w2 // _KERNEL
    hp = (h // 2) // _KERNEL
    w2_v[...] = w2
    wp_v[...] = wp
    m_v[...] = hp * wp

    zf = jnp.zeros((_LANES,), jnp.float32)

    def zero_body(tt, carry):
        for vi in range(_DV):
            zero_v[tt, pl.ds(vi * _LANES, _LANES)] = zf
        return carry

    lax.fori_loop(0, _CT, zero_body, 0)

    n = (_NT // _CT - wid + _NW - 1) // _NW

    bv = (wid % 2) * _CT + tok
    ml = plsc.load_gather(m_v, [bv])
    w2l = plsc.load_gather(w2_v, [bv])
    wpl = plsc.load_gather(wp_v, [bv])
    off = bv * _L
    maxm = jnp.max(ml)
    pbase = wid // 2

    def chunk_params(i):
        t0 = pl.multiple_of((wid + i * _NW) * _CT, _CT)
        ps = pbase + i * _LANES
        p = jnp.full((_LANES,), ps, jnp.int32)
        return t0, p, p < ml, ps < maxm

    def fire_gather(i, buf):
        _, p, _, hv = chunk_params(i)

        @pl.when(hv)
        def _():
            r = p // wpl
            c = p - r * wpl
            base = 2 * r * w2l + 2 * c
            lim = _L - 1
            idx_v[buf, pl.ds(0, _LANES)] = (
                jnp.minimum(base + jbit, lim) + off)
            idx_v[buf, pl.ds(_LANES, _LANES)] = (
                jnp.minimum(base + w2l + jbit, lim) + off)
            pltpu.async_copy(hs_ref.at[idx_v.at[buf]], rows_v.at[buf],
                             semg[buf])

    def process(i, buf):
        t0, p, valid, hv = chunk_params(i)

        @pl.when(i >= 2)
        def _():
            pltpu.make_async_copy(out_v.at[buf],
                                  out_ref.at[pl.ds(0, _CT)],
                                  semo[buf]).wait()

        @pl.when(hv)
        def _():
            pltpu.make_async_copy(hs_ref.at[idx_v.at[buf]], rows_v.at[buf],
                                  semg[buf]).wait()
            scale_v[...] = jnp.where(valid, jnp.float32(0.25),
                                     jnp.float32(0.0))

            def tok_body(tt, c2):
                s = plsc.load_gather(
                    scale_v, [jnp.full((_LANES,), tt, jnp.int32)])
                for vi in range(_DV):
                    sl = pl.ds(vi * _LANES, _LANES)
                    acc = ((rows_v[buf, tt, sl]
                            + rows_v[buf, tt + _CT, sl])
                           + (rows_v[buf, tt + 2 * _CT, sl]
                              + rows_v[buf, tt + 3 * _CT, sl]))
                    out_v[buf, tt, sl] = acc * s
                return c2

            lax.fori_loop(0, _CT, tok_body, 0)
            pltpu.async_copy(out_v.at[buf], out_ref.at[pl.ds(t0, _CT)],
                             semo[buf])

        @pl.when(jnp.logical_not(hv))
        def _():
            pltpu.async_copy(zero_v, out_ref.at[pl.ds(t0, _CT)], semo[buf])

    fire_gather(0, 0)

    def outer(i2, carry):
        for buf in (0, 1):
            i = i2 * 2 + buf

            @pl.when(i < n)
            def _():
                @pl.when(i + 1 < n)
                def _():
                    fire_gather(i + 1, 1 - buf)

                process(i, buf)

        return carry

    lax.fori_loop(0, (n + 1) // 2, outer, 0)

    for buf in (0, 1):
        pltpu.make_async_copy(out_v.at[buf], out_ref.at[pl.ds(0, _CT)],
                              semo[buf]).wait()

    nf = _NT // _LANES
    nmine = (nf - wid + _NW - 1) // _NW

    def attn_body(i, carry):
        g = wid + i * _NW
        t0 = pl.multiple_of(g * _LANES, _LANES)
        t = t0 + lanes
        b = t // _MAX_TOKENS
        pp = t - b * _MAX_TOKENS
        ml = plsc.load_gather(m_v, [b])
        attn_v[...] = jnp.where(pp < ml, jnp.float32(1.0), jnp.float32(0.0))
        pltpu.sync_copy(attn_v, attn_ref.at[pl.ds(t0, _LANES)])
        return carry

    lax.fori_loop(0, nmine, attn_body, 0)


def _build():
    mesh = plsc.VectorSubcoreMesh(core_axis_name="c", subcore_axis_name="s")
    return pl.kernel(
        _sc_body,
        out_type=[
            jax.ShapeDtypeStruct((_NT, _D), jnp.float32),
            jax.ShapeDtypeStruct((_NT,), jnp.float32),
        ],
        mesh=mesh,
        compiler_params=pltpu.CompilerParams(needs_layout_passes=False),
        scratch_types=[
            pltpu.VMEM((_B * 3,), jnp.int32),
            pltpu.VMEM((_LANES,), jnp.int32),
            pltpu.VMEM((_LANES,), jnp.int32),
            pltpu.VMEM((_LANES,), jnp.int32),
            pltpu.VMEM((2, 4 * _CT), jnp.int32),
            pltpu.VMEM((2, 4 * _CT, _D), jnp.float32),
            pltpu.VMEM((2, _CT, _D), jnp.float32),
            pltpu.VMEM((_CT, _D), jnp.float32),
            pltpu.VMEM((_LANES,), jnp.float32),
            pltpu.VMEM((_LANES,), jnp.float32),
            pltpu.SemaphoreType.DMA,
            pltpu.SemaphoreType.DMA,
            pltpu.SemaphoreType.DMA,
            pltpu.SemaphoreType.DMA,
        ],
    )


def kernel(hidden_states, attention_mask, image_grid_thw):
    B, L, D = hidden_states.shape
    assert (B, L, D) == (_B, _L, _D)
    hs_flat = hidden_states.reshape(B * L, D)
    grid_flat = jnp.asarray(image_grid_thw).astype(jnp.int32).reshape(-1)
    out_pm, attn_flat = _build()(hs_flat, grid_flat)
    outputs = out_pm.reshape(_MAX_TOKENS, B, D).transpose(1, 0, 2)
    outputs_attention = attn_flat.reshape(B, _MAX_TOKENS).astype(
        attention_mask.dtype)
    return outputs, outputs_attention
